# baseline (speedup 1.0000x reference)
import functools

import jax
import jax.numpy as jnp
from jax import lax
from jax.experimental import pallas as pl
from jax.experimental.pallas import tpu as pltpu
from jax.experimental.pallas import tpu_sc as plsc

B = 1024
L = 50
D = 512
LPAD = 64
NSEG = 3

NC = 2
NS = 16
LANES = 16
NW = NC * NS

TOK = B * L
BPW = B // NW
C = 56
NCHUNK = BPW
LSTR = 56
PER_W = BPW * LSTR
CPAD = 64
VPR = D // LANES


def _comb_body(pos_ref, seg_ref, out_ref):
    p = pos_ref[:L, :]
    for s in range(NSEG):
        out_ref[s, :L, :] = p + seg_ref[s, :][None, :]


_comb_call = pl.pallas_call(
    _comb_body,
    out_shape=jax.ShapeDtypeStruct((NSEG, LPAD, D), jnp.float32),
)


_sc_mesh = plsc.VectorSubcoreMesh(core_axis_name="c", subcore_axis_name="s")


@functools.partial(
    pl.kernel,
    mesh=_sc_mesh,
    out_type=jax.ShapeDtypeStruct((B * LSTR, D), jnp.float32),
    scratch_types=[
        pltpu.VMEM((PER_W + CPAD,), jnp.int32),
        pltpu.VMEM((PER_W + CPAD,), jnp.int32),
        pltpu.VMEM((CPAD,), jnp.int32),
        pltpu.VMEM((CPAD,), jnp.int32),
        pltpu.VMEM((C, D), jnp.float32),
        pltpu.VMEM((C, D), jnp.float32),
        pltpu.VMEM((C, D), jnp.float32),
        pltpu.VMEM((C, D), jnp.float32),
        pltpu.SemaphoreType.DMA,
        pltpu.SemaphoreType.DMA,
        pltpu.SemaphoreType.DMA,
        pltpu.SemaphoreType.DMA,
        pltpu.SemaphoreType.DMA,
        pltpu.SemaphoreType.DMA,
    ],
)
def _sc_embed(tok_hbm, comb_hbm, seq_hbm, seg_hbm, out_hbm,
              seqv, segv, cidx0, cidx1, tok0, tok1, cmb0, cmb1,
              st0, st1, sc0, sc1, sw0, sw1):
    cidx = (cidx0, cidx1)
    tokb = (tok0, tok1)
    cmbb = (cmb0, cmb1)
    semt = (st0, st1)
    semc = (sc0, sc1)
    semw = (sw0, sw1)

    wid = lax.axis_index("s") * NC + lax.axis_index("c")
    base = wid * PER_W
    bbase = wid * BPW
    pltpu.sync_copy(seq_hbm.at[pl.ds(base, PER_W)], seqv.at[pl.ds(0, PER_W)])
    pltpu.sync_copy(seg_hbm.at[pl.ds(base, PER_W)], segv.at[pl.ds(0, PER_W)])
    del base

    def start_gathers(ic, b):
        off = ic * LSTR
        for j in range(CPAD // LANES):
            lo = off + j * LANES
            lpos = j * LANES + lax.iota(jnp.int32, LANES)
            s16 = segv[pl.ds(lo, LANES)]
            cidx[b][pl.ds(j * LANES, LANES)] = s16 * LPAD + lpos
        pltpu.async_copy(tok_hbm.at[seqv.at[pl.ds(off, C)]], tokb[b], semt[b])
        pltpu.async_copy(comb_hbm.at[cidx[b].at[pl.ds(0, C)]], cmbb[b], semc[b])

    def wait_gathers(ic, b):
        off = ic * LSTR
        pltpu.make_async_copy(tok_hbm.at[seqv.at[pl.ds(off, C)]], tokb[b], semt[b]).wait()
        pltpu.make_async_copy(comb_hbm.at[cidx[b].at[pl.ds(0, C)]], cmbb[b], semc[b]).wait()

    def wait_writeout(ic, b):
        pltpu.make_async_copy(tokb[b], out_hbm.at[pl.ds((bbase + ic) * LSTR, C)], semw[b]).wait()

    start_gathers(0, 0)

    def pair_body(ic2, carry):
        for b in range(2):
            ic = ic2 * 2 + b
            nb = 1 - b

            @pl.when(ic + 1 < NCHUNK)
            def _():
                @pl.when(ic >= 1)
                def _():
                    wait_writeout(ic - 1, nb)
                start_gathers(ic + 1, nb)

            wait_gathers(ic, b)

            def add_body(r, carry2):
                for v in range(VPR):
                    sl = pl.ds(v * LANES, LANES)
                    tokb[b][r, sl] = tokb[b][r, sl] + cmbb[b][r, sl]
                return carry2

            lax.fori_loop(0, C, add_body, 0)

            pltpu.async_copy(tokb[b], out_hbm.at[pl.ds((bbase + ic) * LSTR, C)], semw[b])
        return carry

    lax.fori_loop(0, NCHUNK // 2, pair_body, 0)

    wait_writeout(NCHUNK - 2, 0)
    wait_writeout(NCHUNK - 1, 1)


def kernel(seq, seg_label, token_table, pos_table, seg_table):
    comb = _comb_call(pos_table, seg_table).reshape(NSEG * LPAD, D)
    pad = ((0, 0), (0, LSTR - L))
    seqf = jnp.pad(seq.astype(jnp.int32), pad).reshape(B * LSTR)
    segf = jnp.pad(seg_label.astype(jnp.int32), pad).reshape(B * LSTR)
    out = _sc_embed(token_table, comb, seqf, segf)
    return out.reshape(B, LSTR, D)[:, :L, :]


# speedup vs baseline: 1.0091x; 1.0091x over previous
import functools

import jax
import jax.numpy as jnp
from jax import lax
from jax.experimental import pallas as pl
from jax.experimental.pallas import tpu as pltpu
from jax.experimental.pallas import tpu_sc as plsc

B = 1024
L = 50
D = 512
LPAD = 64
NSEG = 3

NC = 2
NS = 16
LANES = 16
NW = NC * NS

TOK = B * L
BPW = B // NW
C = 32
LSTR = 56
PER_W = BPW * LSTR
NCHUNK = PER_W // C
CPAD = 64
VPR = D // LANES


def _comb_body(pos_ref, seg_ref, out_ref):
    p = pos_ref[:L, :]
    for s in range(NSEG):
        out_ref[s, :L, :] = p + seg_ref[s, :][None, :]


_comb_call = pl.pallas_call(
    _comb_body,
    out_shape=jax.ShapeDtypeStruct((NSEG, LPAD, D), jnp.float32),
)


_sc_mesh = plsc.VectorSubcoreMesh(core_axis_name="c", subcore_axis_name="s")


@functools.partial(
    pl.kernel,
    mesh=_sc_mesh,
    out_type=jax.ShapeDtypeStruct((B * LSTR, D), jnp.float32),
    scratch_types=[
        pltpu.VMEM((PER_W + CPAD,), jnp.int32),
        pltpu.VMEM((PER_W + CPAD,), jnp.int32),
        pltpu.VMEM((CPAD,), jnp.int32),
        pltpu.VMEM((CPAD,), jnp.int32),
        pltpu.VMEM((C, D), jnp.float32),
        pltpu.VMEM((C, D), jnp.float32),
        pltpu.VMEM((C, D), jnp.float32),
        pltpu.VMEM((C, D), jnp.float32),
        pltpu.SemaphoreType.DMA,
        pltpu.SemaphoreType.DMA,
        pltpu.SemaphoreType.DMA,
        pltpu.SemaphoreType.DMA,
        pltpu.SemaphoreType.DMA,
        pltpu.SemaphoreType.DMA,
    ],
)
def _sc_embed(tok_hbm, comb_hbm, seq_hbm, seg_hbm, out_hbm,
              seqv, segv, cidx0, cidx1, tok0, tok1, cmb0, cmb1,
              st0, st1, sc0, sc1, sw0, sw1):
    cidx = (cidx0, cidx1)
    tokb = (tok0, tok1)
    cmbb = (cmb0, cmb1)
    semt = (st0, st1)
    semc = (sc0, sc1)
    semw = (sw0, sw1)

    wid = lax.axis_index("s") * NC + lax.axis_index("c")
    base = wid * PER_W
    bbase = wid * BPW
    pltpu.sync_copy(seq_hbm.at[pl.ds(base, PER_W)], seqv.at[pl.ds(0, PER_W)])
    pltpu.sync_copy(seg_hbm.at[pl.ds(base, PER_W)], segv.at[pl.ds(0, PER_W)])

    def start_gathers(ic, b):
        off = ic * C
        for j in range(C // LANES):
            lo = off + j * LANES
            flat = lo + lax.iota(jnp.int32, LANES)
            lpos = lax.rem(flat, jnp.int32(LSTR))
            s16 = segv[pl.ds(lo, LANES)]
            cidx[b][pl.ds(j * LANES, LANES)] = s16 * LPAD + lpos
        pltpu.async_copy(tok_hbm.at[seqv.at[pl.ds(off, C)]], tokb[b], semt[b])
        pltpu.async_copy(comb_hbm.at[cidx[b].at[pl.ds(0, C)]], cmbb[b], semc[b])

    def wait_gathers(ic, b):
        off = ic * C
        pltpu.make_async_copy(tok_hbm.at[seqv.at[pl.ds(off, C)]], tokb[b], semt[b]).wait()
        pltpu.make_async_copy(comb_hbm.at[cidx[b].at[pl.ds(0, C)]], cmbb[b], semc[b]).wait()

    def wait_writeout(ic, b):
        pltpu.make_async_copy(tokb[b], out_hbm.at[pl.ds(base + ic * C, C)], semw[b]).wait()

    start_gathers(0, 0)

    def pair_body(ic2, carry):
        for b in range(2):
            ic = ic2 * 2 + b
            nb = 1 - b

            @pl.when(ic + 1 < NCHUNK)
            def _():
                @pl.when(ic >= 1)
                def _():
                    wait_writeout(ic - 1, nb)
                start_gathers(ic + 1, nb)

            wait_gathers(ic, b)

            def add_body(r, carry2):
                for v in range(VPR):
                    sl = pl.ds(v * LANES, LANES)
                    tokb[b][r, sl] = tokb[b][r, sl] + cmbb[b][r, sl]
                return carry2

            lax.fori_loop(0, C, add_body, 0)

            pltpu.async_copy(tokb[b], out_hbm.at[pl.ds(base + ic * C, C)], semw[b])
        return carry

    lax.fori_loop(0, NCHUNK // 2, pair_body, 0)

    wait_writeout(NCHUNK - 2, 0)
    wait_writeout(NCHUNK - 1, 1)


def kernel(seq, seg_label, token_table, pos_table, seg_table):
    comb = _comb_call(pos_table, seg_table).reshape(NSEG * LPAD, D)
    pad = ((0, 0), (0, LSTR - L))
    seqf = jnp.pad(seq.astype(jnp.int32), pad).reshape(B * LSTR)
    segf = jnp.pad(seg_label.astype(jnp.int32), pad).reshape(B * LSTR)
    out = _sc_embed(token_table, comb, seqf, segf)
    return out.reshape(B, LSTR, D)[:, :L, :]


# speedup vs baseline: 1.7832x; 1.7672x over previous
import functools

import jax
import jax.numpy as jnp
from jax import lax
from jax.experimental import pallas as pl
from jax.experimental.pallas import tpu as pltpu
from jax.experimental.pallas import tpu_sc as plsc

B = 1024
L = 50
D = 512
LPAD = 64
NSEG = 3

NC = 2
NS = 16
LANES = 16
NW = NC * NS

TOK = B * L
PER_W = TOK // NW
C = 16
NBUF = 4
NCHUNK = PER_W // C
VPR = D // LANES


def _comb_body(pos_ref, seg_ref, out_ref):
    p = pos_ref[:L, :]
    for s in range(NSEG):
        out_ref[s, :L, :] = p + seg_ref[s, :][None, :]


_comb_call = pl.pallas_call(
    _comb_body,
    out_shape=jax.ShapeDtypeStruct((NSEG, LPAD, D), jnp.float32),
)


_sc_mesh = plsc.VectorSubcoreMesh(core_axis_name="c", subcore_axis_name="s")


@functools.partial(
    pl.kernel,
    mesh=_sc_mesh,
    out_type=jax.ShapeDtypeStruct((TOK, D), jnp.float32),
    scratch_types=[
        pltpu.VMEM((PER_W,), jnp.int32),
        pltpu.VMEM((PER_W,), jnp.int32),
    ] + [pltpu.VMEM((C,), jnp.int32)] * NBUF
      + [pltpu.VMEM((C, D), jnp.float32)] * NBUF
      + [pltpu.VMEM((C, D), jnp.float32)] * NBUF
      + [pltpu.SemaphoreType.DMA] * (3 * NBUF),
)
def _sc_embed(tok_hbm, comb_hbm, seq_hbm, seg_hbm, out_hbm,
              seqv, segv, *bufs):
    cidx = bufs[0:NBUF]
    tokb = bufs[NBUF:2 * NBUF]
    cmbb = bufs[2 * NBUF:3 * NBUF]
    semt = bufs[3 * NBUF:4 * NBUF]
    semc = bufs[4 * NBUF:5 * NBUF]
    semw = bufs[5 * NBUF:6 * NBUF]

    wid = lax.axis_index("s") * NC + lax.axis_index("c")
    base = wid * PER_W
    pltpu.sync_copy(seq_hbm.at[pl.ds(base, PER_W)], seqv)
    pltpu.sync_copy(seg_hbm.at[pl.ds(base, PER_W)], segv)

    def start_gathers(ic, b):
        off = ic * C
        for j in range(C // LANES):
            lo = off + j * LANES
            flat = base + lo + lax.iota(jnp.int32, LANES)
            lpos = lax.rem(flat, jnp.int32(L))
            s16 = segv[pl.ds(lo, LANES)]
            cidx[b][pl.ds(j * LANES, LANES)] = s16 * LPAD + lpos
        pltpu.async_copy(tok_hbm.at[seqv.at[pl.ds(off, C)]], tokb[b], semt[b])
        pltpu.async_copy(comb_hbm.at[cidx[b]], cmbb[b], semc[b])

    def wait_gathers(ic, b):
        off = ic * C
        pltpu.make_async_copy(tok_hbm.at[seqv.at[pl.ds(off, C)]], tokb[b], semt[b]).wait()
        pltpu.make_async_copy(comb_hbm.at[cidx[b]], cmbb[b], semc[b]).wait()

    def wait_writeout(ic, b):
        off = ic * C
        pltpu.make_async_copy(tokb[b], out_hbm.at[pl.ds(base + off, C)], semw[b]).wait()

    for p in range(NBUF - 1):
        start_gathers(p, p)

    def ring_body(icq, carry):
        for b in range(NBUF):
            ic = icq * NBUF + b
            pb = (b + NBUF - 1) % NBUF

            @pl.when(ic + NBUF - 1 < NCHUNK)
            def _():
                @pl.when(ic >= 1)
                def _():
                    wait_writeout(ic - 1, pb)
                start_gathers(ic + NBUF - 1, pb)

            wait_gathers(ic, b)

            def add_body(r, carry2):
                for v in range(VPR):
                    sl = pl.ds(v * LANES, LANES)
                    tokb[b][r, sl] = tokb[b][r, sl] + cmbb[b][r, sl]
                return carry2

            lax.fori_loop(0, C, add_body, 0)

            pltpu.async_copy(tokb[b], out_hbm.at[pl.ds(base + ic * C, C)], semw[b])
        return carry

    lax.fori_loop(0, NCHUNK // NBUF, ring_body, 0)

    for q in range(NBUF):
        ic = NCHUNK - NBUF + q
        wait_writeout(ic, ic % NBUF)


def kernel(seq, seg_label, token_table, pos_table, seg_table):
    comb = _comb_call(pos_table, seg_table).reshape(NSEG * LPAD, D)
    seqf = seq.reshape(TOK).astype(jnp.int32)
    segf = seg_label.reshape(TOK).astype(jnp.int32)
    out = _sc_embed(token_table, comb, seqf, segf)
    return out.reshape(B, L, D)
